# Initial kernel scaffold; baseline (speedup 1.0000x reference)
import jax
import jax.numpy as jnp
from jax.experimental import pallas as pl


def kernel(pred_logits, pred_boxes, target_sizes, positive_map):
    raise NotImplementedError("write your pallas kernel here")


# speedup vs baseline: 1.3774x; 1.3774x over previous
import jax
import jax.numpy as jnp
from jax.experimental import pallas as pl
from jax.experimental.pallas import tpu as pltpu

B = 8
Q = 900
T = 512
C = 400
K = 50
BIG = 1 << 30


def _body(logits_ref, boxes_ref, ts_ref, wt_ref,
          scores_ref, labels_ref, boxesout_ref,
          p_ref, bx_ref):
    x = logits_ref[0]
    sig = jax.nn.sigmoid(x).astype(jnp.bfloat16)
    p = jax.lax.dot_general(
        sig, wt_ref[...], (((1,), (0,)), ((), ())),
        preferred_element_type=jnp.float32)
    p_ref[...] = p
    rm = jnp.max(p, axis=1).reshape(1, Q)

    pb = boxes_ref[0]
    cxy = pb[:, 0:2]
    wh2 = pb[:, 2:4] * 0.5
    xyxy = jnp.concatenate([cxy - wh2, cxy + wh2], axis=1)
    ts = ts_ref[0]
    sw = jnp.concatenate([ts[:, 1:2], ts[:, 0:1]], axis=1)
    bx_ref[...] = xyxy * jnp.concatenate([sw, sw], axis=1)

    qio = jax.lax.broadcasted_iota(jnp.int32, (1, Q), 1)
    cio = jax.lax.broadcasted_iota(jnp.int32, (1, C), 1)
    kio = jax.lax.broadcasted_iota(jnp.int32, (1, K), 1)
    sc_acc = jnp.zeros((1, K), jnp.float32)
    lb_acc = jnp.zeros((1, K), jnp.int32)

    for k in range(K):
        m = jnp.max(rm)
        q = jnp.min(jnp.where(rm == m, qio, BIG))
        row = p_ref[pl.ds(q, 1), :]
        c = jnp.min(jnp.where(row == m, cio, BIG))
        sc_acc = jnp.where(kio == k, m, sc_acc)
        lb_acc = jnp.where(kio == k, c, lb_acc)
        boxesout_ref[0, k:k + 1, :] = bx_ref[pl.ds(q, 1), :]
        nrow = jnp.where(cio == c, -1.0, row)
        p_ref[pl.ds(q, 1), :] = nrow
        rm = jnp.where(qio == q, jnp.max(nrow), rm)

    scores_ref[0] = sc_acc
    labels_ref[0] = lb_acc


def kernel(pred_logits, pred_boxes, target_sizes, positive_map):
    grid = (B,)
    scores, labels, boxes = pl.pallas_call(
        _body,
        grid=grid,
        in_specs=[
            pl.BlockSpec((1, Q, T), lambda b: (b, 0, 0)),
            pl.BlockSpec((1, Q, 4), lambda b: (b, 0, 0)),
            pl.BlockSpec((1, 1, 2), lambda b: (b, 0, 0)),
            pl.BlockSpec((T, C), lambda b: (0, 0)),
        ],
        out_specs=[
            pl.BlockSpec((1, 1, K), lambda b: (b, 0, 0)),
            pl.BlockSpec((1, 1, K), lambda b: (b, 0, 0)),
            pl.BlockSpec((1, K, 4), lambda b: (b, 0, 0)),
        ],
        out_shape=[
            jax.ShapeDtypeStruct((B, 1, K), jnp.float32),
            jax.ShapeDtypeStruct((B, 1, K), jnp.int32),
            jax.ShapeDtypeStruct((B, K, 4), jnp.float32),
        ],
        scratch_shapes=[
            pltpu.VMEM((Q, C), jnp.float32),
            pltpu.VMEM((Q, 4), jnp.float32),
        ],
    )(pred_logits, pred_boxes, target_sizes.reshape(B, 1, 2),
      positive_map.T.astype(jnp.bfloat16))
    return (scores.reshape(B, K), labels.reshape(B, K), boxes)


# speedup vs baseline: 5.6043x; 4.0687x over previous
import jax
import jax.numpy as jnp
from jax.experimental import pallas as pl
from jax.experimental.pallas import tpu as pltpu

B = 8
Q = 900
T = 512
C = 400
K = 50
BIG = 1 << 30


def _body(logits_ref, boxes_ref, ts_ref, wt_ref,
          scores_ref, labels_ref, boxesout_ref,
          p_ref, rm_ref, bx_ref):
    pid = pl.program_id(0)

    @pl.when(pid < B)
    def _matmul_step():
        x = logits_ref[0]
        sig = jax.nn.sigmoid(x).astype(jnp.bfloat16)
        p = jax.lax.dot_general(
            sig, wt_ref[...], (((1,), (0,)), ((), ())),
            preferred_element_type=jnp.float32)
        p_ref[pid] = p
        rm_ref[pid] = jnp.max(p, axis=1)

        pb = boxes_ref[0]
        cxy = pb[:, 0:2]
        wh2 = pb[:, 2:4] * 0.5
        xyxy = jnp.concatenate([cxy - wh2, cxy + wh2], axis=1)
        ts = ts_ref[0]
        sw = jnp.concatenate([ts[:, 1:2], ts[:, 0:1]], axis=1)
        bx_ref[pid] = xyxy * jnp.concatenate([sw, sw], axis=1)

    @pl.when(pid == B)
    def _extract_step():
        qio = jax.lax.broadcasted_iota(jnp.int32, (B, Q), 1)
        cio = jax.lax.broadcasted_iota(jnp.int32, (B, C), 1)
        kio = jax.lax.broadcasted_iota(jnp.int32, (B, K), 1)
        rm = rm_ref[...]
        sc_acc = jnp.zeros((B, K), jnp.float32)
        lb_acc = jnp.zeros((B, K), jnp.int32)

        for k in range(K):
            m = jnp.max(rm, axis=1, keepdims=True)
            qv = jnp.where(rm == m, qio, BIG)
            qvec = jnp.min(qv, axis=1, keepdims=True)
            rows = []
            qs = []
            for b in range(B):
                q_b = jnp.min(qv[b])
                qs.append(q_b)
                rows.append(p_ref[b, pl.ds(q_b, 1), :])
            rows8 = jnp.concatenate(rows, axis=0)
            c8 = jnp.min(jnp.where(rows8 == m, cio, BIG),
                         axis=1, keepdims=True)
            nrow8 = jnp.where(cio == c8, -1.0, rows8)
            for b in range(B):
                p_ref[b, pl.ds(qs[b], 1), :] = nrow8[b:b + 1, :]
                boxesout_ref[b, k:k + 1, :] = bx_ref[b, pl.ds(qs[b], 1), :]
            nm8 = jnp.max(nrow8, axis=1, keepdims=True)
            rm = jnp.where(qio == qvec, nm8, rm)
            sc_acc = jnp.where(kio == k, m, sc_acc)
            lb_acc = jnp.where(kio == k, c8, lb_acc)

        scores_ref[...] = sc_acc
        labels_ref[...] = lb_acc


def kernel(pred_logits, pred_boxes, target_sizes, positive_map):
    grid = (B + 1,)
    scores, labels, boxes = pl.pallas_call(
        _body,
        grid=grid,
        in_specs=[
            pl.BlockSpec((1, Q, T), lambda b: (jnp.minimum(b, B - 1), 0, 0)),
            pl.BlockSpec((1, Q, 4), lambda b: (jnp.minimum(b, B - 1), 0, 0)),
            pl.BlockSpec((1, 1, 2), lambda b: (jnp.minimum(b, B - 1), 0, 0)),
            pl.BlockSpec((T, C), lambda b: (0, 0)),
        ],
        out_specs=[
            pl.BlockSpec((B, K), lambda b: (0, 0)),
            pl.BlockSpec((B, K), lambda b: (0, 0)),
            pl.BlockSpec((B, K, 4), lambda b: (0, 0, 0)),
        ],
        out_shape=[
            jax.ShapeDtypeStruct((B, K), jnp.float32),
            jax.ShapeDtypeStruct((B, K), jnp.int32),
            jax.ShapeDtypeStruct((B, K, 4), jnp.float32),
        ],
        scratch_shapes=[
            pltpu.VMEM((B, Q, C), jnp.float32),
            pltpu.VMEM((B, Q), jnp.float32),
            pltpu.VMEM((B, Q, 4), jnp.float32),
        ],
    )(pred_logits, pred_boxes, target_sizes.reshape(B, 1, 2),
      positive_map.T.astype(jnp.bfloat16))
    return (scores, labels, boxes)
